# baseline (speedup 1.0000x reference)
import functools

import jax
import jax.numpy as jnp
from jax import lax
from jax.experimental import pallas as pl
from jax.experimental.pallas import tpu as pltpu
from jax.experimental.pallas import tpu_sc as plsc


def _score_body(x_ref, t_ref, score_ref, num_ref, xsum_ref, *, n_out):
    step = pl.program_id(0)
    nsteps = pl.num_programs(0)

    @pl.when(step == 0)
    def _init():
        num_ref[...] = jnp.zeros_like(num_ref)
        xsum_ref[...] = jnp.zeros_like(xsum_ref)

    xb = x_ref[...]
    xsum_ref[...] += xb.sum(axis=0, keepdims=True)
    for o in range(n_out):
        tcol = t_ref[:, o:o + 1]
        contrib = jnp.maximum(xb - tcol, 0.0).sum(axis=0, keepdims=True)
        num_ref[o:o + 1, :] += contrib

    @pl.when(step == nsteps - 1)
    def _finish():
        xs = xsum_ref[...]
        num = num_ref[...]
        score_ref[...] = jnp.where(xs == 0.0, 0.0, num / xs)


def _compute_score(x, t):
    b, f = x.shape
    n_out = t.shape[1]
    bc = 128
    return pl.pallas_call(
        functools.partial(_score_body, n_out=n_out),
        grid=(b // bc,),
        in_specs=[
            pl.BlockSpec((bc, f), lambda i: (i, 0)),
            pl.BlockSpec((bc, n_out), lambda i: (i, 0)),
        ],
        out_specs=pl.BlockSpec((n_out, f), lambda i: (0, 0)),
        out_shape=jax.ShapeDtypeStruct((n_out, f), jnp.float32),
        scratch_shapes=[
            pltpu.VMEM((n_out, f), jnp.float32),
            pltpu.VMEM((1, f), jnp.float32),
        ],
    )(x, t)


def _rank_body(srow_ref, sj_ref, wrow_ref, wj_ref, rs_ref, rw_ref, *,
               jb, f, n_out):
    j = pl.program_id(0)
    o = pl.program_id(1)
    sk = srow_ref[0]
    wk = wrow_ref[0]
    oh = lax.broadcasted_iota(jnp.int32, (1, n_out), 1) == o
    sj = jnp.where(oh, sj_ref[0], 0.0).sum(axis=1, keepdims=True)
    wj = jnp.where(oh, wj_ref[0], 0.0).sum(axis=1, keepdims=True)
    kk = lax.broadcasted_iota(jnp.int32, (jb, f), 1)
    jj = j * jb + lax.broadcasted_iota(jnp.int32, (jb, f), 0)
    kltj = kk < jj
    inc_s = (sk > sj) | ((sk == sj) & kltj)
    inc_w = (wk > wj) | ((wk == wj) & kltj)
    rcol_s = inc_s.astype(jnp.int32).sum(axis=1, keepdims=True)
    rcol_w = inc_w.astype(jnp.int32).sum(axis=1, keepdims=True)
    ohi = oh.astype(jnp.int32)

    @pl.when(o == 0)
    def _init():
        rs_ref[...] = jnp.zeros_like(rs_ref)
        rw_ref[...] = jnp.zeros_like(rw_ref)

    rs_ref[0] += rcol_s * ohi
    rw_ref[0] += rcol_w * ohi


def _compute_ranks(score_t, score_n, w_t, w_n):
    n_out, f = score_t.shape
    jb = 256
    nj = f // jb
    rs3, rw3 = pl.pallas_call(
        functools.partial(_rank_body, jb=jb, f=f, n_out=n_out),
        grid=(nj, n_out),
        in_specs=[
            pl.BlockSpec((1, 1, f), lambda j, o: (o, 0, 0)),
            pl.BlockSpec((1, jb, n_out), lambda j, o: (j, 0, 0)),
            pl.BlockSpec((1, 1, f), lambda j, o: (o, 0, 0)),
            pl.BlockSpec((1, jb, n_out), lambda j, o: (j, 0, 0)),
        ],
        out_specs=[
            pl.BlockSpec((1, jb, n_out), lambda j, o: (j, 0, 0)),
            pl.BlockSpec((1, jb, n_out), lambda j, o: (j, 0, 0)),
        ],
        out_shape=[
            jax.ShapeDtypeStruct((nj, jb, n_out), jnp.int32),
            jax.ShapeDtypeStruct((nj, jb, n_out), jnp.int32),
        ],
    )(score_t.reshape(n_out, 1, f), score_n.reshape(nj, jb, n_out),
      w_t.reshape(n_out, 1, f), w_n.reshape(nj, jb, n_out))
    return rs3.reshape(f, n_out), rw3.reshape(f, n_out)


def _make_sc_pairing(n_out, f):
    info = plsc.get_sparse_core_info()
    nc, ns, lanes = info.num_cores, info.num_subcores, info.num_lanes
    nw = nc * ns
    assert n_out == nw and f % lanes == 0

    mesh = plsc.VectorSubcoreMesh(core_axis_name="c", subcore_axis_name="s")

    @functools.partial(
        pl.kernel,
        out_type=jax.ShapeDtypeStruct((n_out, lanes), jnp.float32),
        mesh=mesh,
        compiler_params=pltpu.CompilerParams(needs_layout_passes=False),
        scratch_types=[
            pltpu.VMEM((f,), jnp.float32),
            pltpu.VMEM((f,), jnp.int32),
            pltpu.VMEM((f,), jnp.int32),
            pltpu.VMEM((f,), jnp.float32),
            pltpu.VMEM((f,), jnp.float32),
            pltpu.VMEM((lanes,), jnp.float32),
        ],
    )
    def sc_pair(w_hbm, rs_hbm, rw_hbm, out_hbm, w_v, rs_v, rw_v, t_v, s_v,
                acc_v):
        wid = lax.axis_index("s") * nc + lax.axis_index("c")
        pltpu.sync_copy(w_hbm.at[wid], w_v)
        pltpu.sync_copy(rs_hbm.at[wid], rs_v)
        pltpu.sync_copy(rw_hbm.at[wid], rw_v)

        def scat(i, carry):
            sl = pl.ds(i * lanes, lanes)
            wv = w_v[sl]
            plsc.store_scatter(t_v, [rs_v[sl]], wv)
            plsc.store_scatter(s_v, [rw_v[sl]], wv)
            return carry

        lax.fori_loop(0, f // lanes, scat, jnp.int32(0))

        def red(i, acc):
            sl = pl.ds(i * lanes, lanes)
            d = s_v[sl] - t_v[sl]
            return acc + d * d

        acc = lax.fori_loop(0, f // lanes, red,
                            jnp.zeros((lanes,), jnp.float32))
        acc_v[...] = acc
        pltpu.sync_copy(acc_v, out_hbm.at[wid])

    return sc_pair


def kernel(x, y, t, w, base_w):
    del y, base_w
    f, n_out = w.shape

    score_t = _compute_score(x, t)
    score_n = score_t.T
    w_t = w.T
    rs_n, rw_n = _compute_ranks(score_t, score_n, w_t, w)
    partial = _make_sc_pairing(n_out, f)(w_t, rs_n.T, rw_n.T)
    return partial.sum() / (f * n_out)


# speedup vs baseline: 3.5493x; 3.5493x over previous
import functools

import jax
import jax.numpy as jnp
from jax import lax
from jax.experimental import pallas as pl
from jax.experimental.pallas import tpu as pltpu
from jax.experimental.pallas import tpu_sc as plsc


def _partner(x, s):
    f = x.shape[-1]
    lanes = lax.broadcasted_iota(jnp.int32, (1, f), 1)
    bit_clear = (lanes & s) == 0
    return jnp.where(bit_clear, pltpu.roll(x, f - s, 1), pltpu.roll(x, s, 1))


def _sort_desc_rows(key, idx):
    f = key.shape[-1]
    lanes = lax.broadcasted_iota(jnp.int32, (1, f), 1)
    k = 2
    while k <= f:
        s = k // 2
        while s >= 1:
            pk = _partner(key, s)
            m = ((lanes & s) == 0) ^ ((lanes & k) == 0)
            if idx is not None:
                pi = _partner(idx, s)
                mine_first = (key > pk) | ((key == pk) & (idx < pi))
                take = mine_first ^ jnp.logical_not(m)
                key = jnp.where(take, pk, key)
                idx = jnp.where(take, pi, idx)
            else:
                key = jnp.where(m, jnp.minimum(key, pk),
                                jnp.maximum(key, pk))
            s //= 2
        k *= 2
    return key, idx


def _score_sort_body(x_ref, t_ref, w_t_ref, idx_ref, ws_ref, num_ref,
                     xsum_ref, *, n_out):
    step = pl.program_id(0)
    nsteps = pl.num_programs(0)

    @pl.when(step == 0)
    def _init():
        num_ref[...] = jnp.zeros_like(num_ref)
        xsum_ref[...] = jnp.zeros_like(xsum_ref)

    xb = x_ref[...]
    xsum_ref[...] += xb.sum(axis=0, keepdims=True)
    for o in range(n_out):
        tcol = t_ref[:, o:o + 1]
        contrib = jnp.maximum(xb - tcol, 0.0).sum(axis=0, keepdims=True)
        num_ref[o:o + 1, :] += contrib

    @pl.when(step == nsteps - 1)
    def _finish():
        xs = xsum_ref[...]
        score = jnp.where(xs == 0.0, 0.0, num_ref[...] / xs)
        iota = lax.broadcasted_iota(jnp.int32, score.shape, 1)
        _, sidx = _sort_desc_rows(score, iota)
        idx_ref[...] = sidx
        ws, _ = _sort_desc_rows(w_t_ref[...], None)
        ws_ref[...] = ws


def _score_and_sort(x, t, w_t):
    b, f = x.shape
    n_out = t.shape[1]
    bc = 128
    return pl.pallas_call(
        functools.partial(_score_sort_body, n_out=n_out),
        grid=(b // bc,),
        in_specs=[
            pl.BlockSpec((bc, f), lambda i: (i, 0)),
            pl.BlockSpec((bc, n_out), lambda i: (i, 0)),
            pl.BlockSpec((n_out, f), lambda i: (0, 0)),
        ],
        out_specs=[
            pl.BlockSpec((n_out, f), lambda i: (0, 0)),
            pl.BlockSpec((n_out, f), lambda i: (0, 0)),
        ],
        out_shape=[
            jax.ShapeDtypeStruct((n_out, f), jnp.int32),
            jax.ShapeDtypeStruct((n_out, f), jnp.float32),
        ],
        scratch_shapes=[
            pltpu.VMEM((n_out, f), jnp.float32),
            pltpu.VMEM((1, f), jnp.float32),
        ],
    )(x, t, w_t)


def _make_sc_pairing(n_out, f):
    info = plsc.get_sparse_core_info()
    nc, ns, lanes = info.num_cores, info.num_subcores, info.num_lanes
    assert n_out == nc * ns and f % lanes == 0

    mesh = plsc.VectorSubcoreMesh(core_axis_name="c", subcore_axis_name="s")

    @functools.partial(
        pl.kernel,
        out_type=jax.ShapeDtypeStruct((n_out, lanes), jnp.float32),
        mesh=mesh,
        compiler_params=pltpu.CompilerParams(needs_layout_passes=False),
        scratch_types=[
            pltpu.VMEM((f,), jnp.float32),
            pltpu.VMEM((f,), jnp.int32),
            pltpu.VMEM((f,), jnp.float32),
            pltpu.VMEM((lanes,), jnp.float32),
        ],
    )
    def sc_pair(w_hbm, idx_hbm, ws_hbm, out_hbm, w_v, idx_v, ws_v, acc_v):
        wid = lax.axis_index("s") * nc + lax.axis_index("c")
        pltpu.sync_copy(w_hbm.at[wid], w_v)
        pltpu.sync_copy(idx_hbm.at[wid], idx_v)
        pltpu.sync_copy(ws_hbm.at[wid], ws_v)

        def red(i, acc):
            sl = pl.ds(i * lanes, lanes)
            tv = plsc.load_gather(w_v, [idx_v[sl]])
            d = ws_v[sl] - tv
            return acc + d * d

        acc = lax.fori_loop(0, f // lanes, red,
                            jnp.zeros((lanes,), jnp.float32))
        acc_v[...] = acc
        pltpu.sync_copy(acc_v, out_hbm.at[wid])

    return sc_pair


def kernel(x, y, t, w, base_w):
    del y, base_w
    f, n_out = w.shape

    w_t = w.T
    idx_t, ws_t = _score_and_sort(x, t, w_t)
    partial = _make_sc_pairing(n_out, f)(w_t, idx_t, ws_t)
    return partial.sum() / (f * n_out)
